# baseline (speedup 1.0000x reference)
import functools

import jax
import jax.numpy as jnp
from jax import lax
from jax.experimental import pallas as pl
from jax.experimental.pallas import tpu as pltpu
from jax.experimental.pallas import tpu_sc as plsc

N = 4096
D = 128
K1 = 30
K2 = 6
BETA1 = 0.2
BETA2 = 0.2
SCALE = 0.3
BLK = 256
NSTEPS = N // BLK
W1 = 32
W2 = 8
SLOTS = W1 + W2
NEG = -3.0


CH = 32
CW = 128
T1 = 8


def _topk_body(xb_ref, xall_ref, labr_ref, labc_ref, ew_ref, idx_ref, dc_ref):
    step = pl.program_id(0)
    base = step * BLK
    xb = xb_ref[...]
    xall = xall_ref[...]
    sim = lax.dot_general(xb, xall, (((1,), (1,)), ((), ())),
                          preferred_element_type=jnp.float32)
    cols = lax.broadcasted_iota(jnp.int32, (BLK, N), 1)
    rid = base + lax.broadcasted_iota(jnp.int32, (BLK, 1), 0)
    diag = cols == rid

    def extract(work3, T):
        ccols = lax.broadcasted_iota(jnp.int32, (BLK, CH, CW), 2).astype(
            jnp.float32)
        tslot = lax.broadcasted_iota(jnp.int32, (BLK, CH, T), 2)

        def body(j, carry):
            w3, tv, tc = carry
            m = jnp.max(w3, axis=2, keepdims=True)
            eq = w3 == m
            c = jnp.min(jnp.where(eq, ccols, jnp.float32(CW)), axis=2,
                        keepdims=True)
            tv = jnp.where(tslot == j, m, tv)
            tc = jnp.where(tslot == j, c, tc)
            w3 = jnp.where(eq, NEG, w3)
            return w3, tv, tc

        tv0 = jnp.full((BLK, CH, T), -1e30, jnp.float32)
        tc0 = jnp.zeros((BLK, CH, T), jnp.float32)
        _, tv, tc = lax.fori_loop(0, T, body, (work3, tv0, tc0))
        return tv, tc

    def summary_select(sv, tc, k, width):
        chunkid = lax.broadcasted_iota(jnp.int32, (BLK, CH, T1), 1).astype(
            jnp.float32)
        gcol = chunkid * CW + tc
        wcols = lax.broadcasted_iota(jnp.int32, (BLK, width), 1)

        def body(t, carry):
            sv, vals, idxs = carry
            m = jnp.max(jnp.max(sv, axis=2), axis=1, keepdims=True)
            eq = sv == m[:, :, None]
            masked_col = jnp.where(eq, gcol, jnp.float32(N))
            sel = jnp.min(jnp.min(masked_col, axis=2), axis=1, keepdims=True)
            slot = wcols == t
            vals = jnp.where(slot, m, vals)
            idxs = jnp.where(slot, sel, idxs)
            sv = jnp.where(eq, NEG, sv)
            return sv, vals, idxs

        vals0 = jnp.full((BLK, width), -1e30, jnp.float32)
        idxs0 = jnp.zeros((BLK, width), jnp.float32)
        svf, vals, idxs = lax.fori_loop(0, k, body, (sv, vals0, idxs0))
        return svf, vals, idxs.astype(jnp.int32)

    def select_exact(work0, k, width):
        wcols = lax.broadcasted_iota(jnp.int32, (BLK, width), 1)

        def body(t, carry):
            work, vals, idxs = carry
            m = jnp.max(work, axis=1, keepdims=True)
            eq = work == m
            sel = jnp.min(jnp.where(eq, cols, N), axis=1, keepdims=True)
            slot = wcols == t
            vals = jnp.where(slot, m, vals)
            idxs = jnp.where(slot, sel, idxs)
            work = jnp.where(eq, NEG, work)
            return work, vals, idxs

        vals0 = jnp.full((BLK, width), -1e30, jnp.float32)
        idxs0 = jnp.zeros((BLK, width), jnp.int32)
        _, vals, idxs = lax.fori_loop(0, k, body, (work0, vals0, idxs0))
        return vals, idxs

    tv1, tc1 = extract(sim.reshape(BLK, CH, CW), T1)
    svf1, vals1f, idx1f = summary_select(tv1, tc1, K1, W1)
    t30 = jnp.min(vals1f[:, :K1], axis=1, keepdims=True)
    consumed = jnp.sum((svf1 == NEG).astype(jnp.int32), axis=2)
    flag = jnp.any((consumed >= T1) & (tv1[:, :, T1 - 1] > t30))
    vals1, idx1 = lax.cond(flag,
                           lambda: select_exact(sim, K1, W1),
                           lambda: (vals1f, idx1f))
    t1 = jnp.min(vals1[:, :K1], axis=1, keepdims=True)
    es = jnp.exp(sim / BETA1)
    e1 = jnp.where(sim >= t1, es, 0.0)
    rowsum1 = jnp.sum(e1, axis=1, keepdims=True)
    ew1 = ((1.0 - SCALE) * lax.rsqrt(rowsum1)) * jnp.exp(vals1 / BETA1)

    same_cam = labr_ref[...] == labc_ref[...]
    mask2 = same_cam & (~diag)
    sim2 = jnp.where(mask2, -2.0, sim)
    tv2, tc2 = extract(sim2.reshape(BLK, CH, CW), T1)
    _, vals2, idx2 = summary_select(tv2, tc2, K2, W2)
    t2 = jnp.min(vals2[:, :K2], axis=1, keepdims=True)
    e2 = jnp.where(sim2 >= t2,
                   jnp.where(mask2, jnp.float32(4.5399929762484854e-05), es),
                   0.0)
    e2 = jnp.where(diag, es, e2)
    rowsum2 = jnp.sum(e2, axis=1, keepdims=True)
    dr2 = SCALE * lax.rsqrt(rowsum2)
    ew2 = dr2 * jnp.exp(vals2 / BETA2)
    dvec = jnp.sum(jnp.where(diag, sim, 0.0), axis=1, keepdims=True)
    has_diag = jnp.any(idx2[:, :K2] == rid, axis=1, keepdims=True)
    ew_diag = jnp.where(has_diag, 0.0, dr2 * jnp.exp(dvec / BETA2))
    ew2_full = jnp.concatenate(
        [ew2[:, :K2], ew_diag, jnp.zeros((BLK, W2 - K2 - 1), jnp.float32)], axis=1)
    idx2_full = jnp.concatenate(
        [idx2[:, :K2], rid, jnp.zeros((BLK, W2 - K2 - 1), jnp.int32)], axis=1) + N

    ew_ref[...] = jnp.concatenate([ew1, ew2_full], axis=1)
    idx_ref[...] = jnp.concatenate([idx1, idx2_full], axis=1)

    cp = jnp.concatenate([jnp.sum(e1, axis=0, keepdims=True),
                          jnp.sum(e2, axis=0, keepdims=True)], axis=0)

    @pl.when(step == 0)
    def _init():
        dc_ref[...] = cp

    @pl.when(step != 0)
    def _acc():
        dc_ref[...] += cp

    @pl.when(step == NSTEPS - 1)
    def _fin():
        dc_ref[...] = lax.rsqrt(dc_ref[...])


_topk_call = pl.pallas_call(
    _topk_body,
    grid=(NSTEPS,),
    in_specs=[
        pl.BlockSpec((BLK, D), lambda i: (i, 0)),
        pl.BlockSpec((N, D), lambda i: (0, 0)),
        pl.BlockSpec((BLK, 1), lambda i: (i, 0)),
        pl.BlockSpec((1, N), lambda i: (0, 0)),
    ],
    out_specs=[
        pl.BlockSpec((BLK, SLOTS), lambda i: (i, 0)),
        pl.BlockSpec((BLK, SLOTS), lambda i: (i, 0)),
        pl.BlockSpec((2, N), lambda i: (0, 0)),
    ],
    out_shape=[
        jax.ShapeDtypeStruct((N, SLOTS), jnp.float32),
        jax.ShapeDtypeStruct((N, SLOTS), jnp.int32),
        jax.ShapeDtypeStruct((2, N), jnp.float32),
    ],
)


NW = 32
RPW = N // NW


def _sc_row_compute(rb, idx_v, ew_v, dc_v, w_v, bc_v, out_v, r):
    for c in range((SLOTS + 15) // 16):
        idxc = idx_v[pl.ds(r * SLOTS + c * 16, 16)]
        ewc = ew_v[pl.ds(r * SLOTS + c * 16, 16)]
        valid = SLOTS - c * 16
        if valid >= 16:
            dcg = plsc.load_gather(dc_v, [idxc])
        else:
            lanes = lax.broadcasted_iota(jnp.int32, (16,), 0)
            dcg = plsc.load_gather(dc_v, [idxc], mask=lanes < valid)
        w_v[pl.ds((c + 1) * 16, 16)] = ewc * dcg
    accs = [jnp.zeros((16,), jnp.float32) for _ in range(D // 16)]
    for k in range(SLOTS):
        wb = plsc.load_gather(w_v, [jnp.full((16,), 16 + k, jnp.int32)])
        for d in range(D // 16):
            accs[d] = accs[d] + wb * rb[k, pl.ds(d * 16, 16)]
    sq = accs[0] * accs[0]
    for d in range(1, D // 16):
        sq = sq + accs[d] * accs[d]
    bc_v[...] = jnp.cumsum(sq)
    tot = plsc.load_gather(bc_v, [jnp.full((16,), 15, jnp.int32)])
    yi = jnp.int32(0x5F3759DF) - lax.shift_right_logical(
        plsc.bitcast(tot, jnp.int32), 1)
    y = plsc.bitcast(yi, jnp.float32)
    for _ in range(3):
        y = y * (1.5 - 0.5 * tot * y * y)
    for d in range(D // 16):
        out_v[pl.ds(r * D + d * 16, 16)] = accs[d] * y


NBUF = 8


def _sc_agg_body(x2_hbm, idx_hbm, ew_hbm, dc_hbm, out_hbm,
                 idx_v, ew_v, dc_v, rb0, rb1, rb2, rb3, rb4, rb5, rb6, rb7,
                 w_v, bc_v, out_v,
                 sem0, sem1, sem2, sem3, sem4, sem5, sem6, sem7):
    rbs = (rb0, rb1, rb2, rb3, rb4, rb5, rb6, rb7)
    sems = (sem0, sem1, sem2, sem3, sem4, sem5, sem6, sem7)
    wid = lax.axis_index("s") * 2 + lax.axis_index("c")
    base = wid * RPW
    pltpu.sync_copy(idx_hbm.at[pl.ds(base * SLOTS, RPW * SLOTS)],
                    idx_v.at[pl.ds(0, RPW * SLOTS)])
    pltpu.sync_copy(ew_hbm.at[pl.ds(base * SLOTS, RPW * SLOTS)],
                    ew_v.at[pl.ds(0, RPW * SLOTS)])
    pltpu.sync_copy(dc_hbm, dc_v)

    def issue(r, rb, sem):
        pltpu.async_copy(x2_hbm.at[idx_v.at[pl.ds(r * SLOTS, SLOTS)]], rb, sem)

    def wait(rb, sem):
        pltpu.make_async_copy(x2_hbm.at[pl.ds(0, SLOTS)], rb, sem).wait()

    for j in range(NBUF):
        issue(j, rbs[j], sems[j])

    def group(g, carry):
        r0 = g * NBUF
        for j in range(NBUF):
            wait(rbs[j], sems[j])
            _sc_row_compute(rbs[j], idx_v, ew_v, dc_v, w_v, bc_v, out_v, r0 + j)

            @pl.when(r0 + j + NBUF < RPW)
            def _next():
                issue(r0 + j + NBUF, rbs[j], sems[j])

        return carry

    lax.fori_loop(0, RPW // NBUF, group, 0)
    pltpu.sync_copy(out_v, out_hbm.at[pl.ds(base * D, RPW * D)])


@functools.cache
def _get_sc_agg():
    return functools.partial(
        pl.kernel,
        out_type=jax.ShapeDtypeStruct((N * D,), jnp.float32),
        mesh=plsc.VectorSubcoreMesh(core_axis_name="c", subcore_axis_name="s"),
        compiler_params=pltpu.CompilerParams(needs_layout_passes=False),
        scratch_types=[
            pltpu.VMEM((RPW * SLOTS + 16,), jnp.int32),
            pltpu.VMEM((RPW * SLOTS + 16,), jnp.float32),
            pltpu.VMEM((2 * N,), jnp.float32),
            pltpu.VMEM((SLOTS, D), jnp.float32),
            pltpu.VMEM((SLOTS, D), jnp.float32),
            pltpu.VMEM((SLOTS, D), jnp.float32),
            pltpu.VMEM((SLOTS, D), jnp.float32),
            pltpu.VMEM((SLOTS, D), jnp.float32),
            pltpu.VMEM((SLOTS, D), jnp.float32),
            pltpu.VMEM((SLOTS, D), jnp.float32),
            pltpu.VMEM((SLOTS, D), jnp.float32),
            pltpu.VMEM((16 + 48,), jnp.float32),
            pltpu.VMEM((16,), jnp.float32),
            pltpu.VMEM((RPW * D,), jnp.float32),
            pltpu.SemaphoreType.DMA,
            pltpu.SemaphoreType.DMA,
            pltpu.SemaphoreType.DMA,
            pltpu.SemaphoreType.DMA,
            pltpu.SemaphoreType.DMA,
            pltpu.SemaphoreType.DMA,
            pltpu.SemaphoreType.DMA,
            pltpu.SemaphoreType.DMA,
        ],
    )(_sc_agg_body)


def kernel(X, labels_cam):
    lab = labels_cam.astype(jnp.int32)
    ew, idx, dc = _topk_call(X, X, lab.reshape(N, 1), lab.reshape(1, N))
    x2 = jnp.concatenate([X, X], axis=0)
    out = _get_sc_agg()(x2, idx.reshape(-1), ew.reshape(-1), dc.reshape(-1))
    return out.reshape(N, D)


# speedup vs baseline: 2.1494x; 2.1494x over previous
import functools

import jax
import jax.numpy as jnp
from jax import lax
from jax.experimental import pallas as pl
from jax.experimental.pallas import tpu as pltpu
from jax.experimental.pallas import tpu_sc as plsc

N = 4096
D = 128
K1 = 30
K2 = 6
BETA1 = 0.2
BETA2 = 0.2
SCALE = 0.3
BLK = 256
NSTEPS = N // BLK
W1 = 32
W2 = 8
SLOTS = W1 + W2
NEG = -3.0


CH = 32
CW = 128
T1 = 8


def _topk_body(xb_ref, xall_ref, labr_ref, labc_ref, ew_ref, idx_ref, dc_ref):
    step = pl.program_id(0)
    base = step * BLK
    xb = xb_ref[...]
    xall = xall_ref[...]
    sim = lax.dot_general(xb, xall, (((1,), (1,)), ((), ())),
                          preferred_element_type=jnp.float32)
    cols = lax.broadcasted_iota(jnp.int32, (BLK, N), 1)
    rid = base + lax.broadcasted_iota(jnp.int32, (BLK, 1), 0)
    diag = cols == rid

    def select(work0, k, width):
        wcols = lax.broadcasted_iota(jnp.int32, (BLK, width), 1)

        def body(t, carry):
            work, vals, idxs = carry
            m = jnp.max(work, axis=1, keepdims=True)
            eq = work == m
            sel = jnp.min(jnp.where(eq, cols, N), axis=1, keepdims=True)
            slot = wcols == t
            vals = jnp.where(slot, m, vals)
            idxs = jnp.where(slot, sel, idxs)
            work = jnp.where(eq, NEG, work)
            return work, vals, idxs

        vals0 = jnp.full((BLK, width), -1e30, jnp.float32)
        idxs0 = jnp.zeros((BLK, width), jnp.int32)
        return lax.fori_loop(0, k, body, (work0, vals0, idxs0))

    work1, vals1, idx1 = select(sim, K1, W1)
    es = jnp.exp(sim / BETA1)
    e1 = jnp.where(work1 == NEG, es, 0.0)
    rowsum1 = jnp.sum(e1, axis=1, keepdims=True)
    ew1 = ((1.0 - SCALE) * lax.rsqrt(rowsum1)) * jnp.exp(vals1 / BETA1)

    same_cam = labr_ref[...] == labc_ref[...]
    mask2 = same_cam & (~diag)
    sim2 = jnp.where(mask2, -2.0, sim)
    work2, vals2, idx2 = select(sim2, K2, W2)
    e2 = jnp.where(work2 == NEG,
                   jnp.where(mask2, jnp.float32(4.5399929762484854e-05), es),
                   0.0)
    e2 = jnp.where(diag, es, e2)
    rowsum2 = jnp.sum(e2, axis=1, keepdims=True)
    dr2 = SCALE * lax.rsqrt(rowsum2)
    ew2 = dr2 * jnp.exp(vals2 / BETA2)
    dvec = jnp.sum(jnp.where(diag, sim, 0.0), axis=1, keepdims=True)
    has_diag = jnp.any(idx2[:, :K2] == rid, axis=1, keepdims=True)
    ew_diag = jnp.where(has_diag, 0.0, dr2 * jnp.exp(dvec / BETA2))
    ew2_full = jnp.concatenate(
        [ew2[:, :K2], ew_diag, jnp.zeros((BLK, W2 - K2 - 1), jnp.float32)], axis=1)
    idx2_full = jnp.concatenate(
        [idx2[:, :K2], rid, jnp.zeros((BLK, W2 - K2 - 1), jnp.int32)], axis=1) + N

    ew_ref[...] = jnp.concatenate([ew1, ew2_full], axis=1)
    idx_ref[...] = jnp.concatenate([idx1, idx2_full], axis=1)

    cp = jnp.concatenate([jnp.sum(e1, axis=0, keepdims=True),
                          jnp.sum(e2, axis=0, keepdims=True)], axis=0)

    @pl.when(step == 0)
    def _init():
        dc_ref[...] = cp

    @pl.when(step != 0)
    def _acc():
        dc_ref[...] += cp

    @pl.when(step == NSTEPS - 1)
    def _fin():
        dc_ref[...] = lax.rsqrt(dc_ref[...])


_topk_call = pl.pallas_call(
    _topk_body,
    grid=(NSTEPS,),
    in_specs=[
        pl.BlockSpec((BLK, D), lambda i: (i, 0)),
        pl.BlockSpec((N, D), lambda i: (0, 0)),
        pl.BlockSpec((BLK, 1), lambda i: (i, 0)),
        pl.BlockSpec((1, N), lambda i: (0, 0)),
    ],
    out_specs=[
        pl.BlockSpec((BLK, SLOTS), lambda i: (i, 0)),
        pl.BlockSpec((BLK, SLOTS), lambda i: (i, 0)),
        pl.BlockSpec((2, N), lambda i: (0, 0)),
    ],
    out_shape=[
        jax.ShapeDtypeStruct((N, SLOTS), jnp.float32),
        jax.ShapeDtypeStruct((N, SLOTS), jnp.int32),
        jax.ShapeDtypeStruct((2, N), jnp.float32),
    ],
)


NW = 32
RPW = N // NW


def _sc_row_compute(rb, idx_v, ew_v, dc_v, w_v, bc_v, out_v, r):
    for c in range((SLOTS + 15) // 16):
        idxc = idx_v[pl.ds(r * SLOTS + c * 16, 16)]
        ewc = ew_v[pl.ds(r * SLOTS + c * 16, 16)]
        valid = SLOTS - c * 16
        if valid >= 16:
            dcg = plsc.load_gather(dc_v, [idxc])
        else:
            lanes = lax.broadcasted_iota(jnp.int32, (16,), 0)
            dcg = plsc.load_gather(dc_v, [idxc], mask=lanes < valid)
        w_v[pl.ds((c + 1) * 16, 16)] = ewc * dcg
    accs = [jnp.zeros((16,), jnp.float32) for _ in range(D // 16)]
    for k in range(SLOTS):
        wb = plsc.load_gather(w_v, [jnp.full((16,), 16 + k, jnp.int32)])
        for d in range(D // 16):
            accs[d] = accs[d] + wb * rb[k, pl.ds(d * 16, 16)]
    sq = accs[0] * accs[0]
    for d in range(1, D // 16):
        sq = sq + accs[d] * accs[d]
    bc_v[...] = jnp.cumsum(sq)
    tot = plsc.load_gather(bc_v, [jnp.full((16,), 15, jnp.int32)])
    yi = jnp.int32(0x5F3759DF) - lax.shift_right_logical(
        plsc.bitcast(tot, jnp.int32), 1)
    y = plsc.bitcast(yi, jnp.float32)
    for _ in range(3):
        y = y * (1.5 - 0.5 * tot * y * y)
    for d in range(D // 16):
        out_v[pl.ds(r * D + d * 16, 16)] = accs[d] * y


NBUF = 8


def _sc_agg_body(x2_hbm, idx_hbm, ew_hbm, dc_hbm, out_hbm,
                 idx_v, ew_v, dc_v, rb0, rb1, rb2, rb3, rb4, rb5, rb6, rb7,
                 w_v, bc_v, out_v,
                 sem0, sem1, sem2, sem3, sem4, sem5, sem6, sem7):
    rbs = (rb0, rb1, rb2, rb3, rb4, rb5, rb6, rb7)
    sems = (sem0, sem1, sem2, sem3, sem4, sem5, sem6, sem7)
    wid = lax.axis_index("s") * 2 + lax.axis_index("c")
    base = wid * RPW
    pltpu.sync_copy(idx_hbm.at[pl.ds(base * SLOTS, RPW * SLOTS)],
                    idx_v.at[pl.ds(0, RPW * SLOTS)])
    pltpu.sync_copy(ew_hbm.at[pl.ds(base * SLOTS, RPW * SLOTS)],
                    ew_v.at[pl.ds(0, RPW * SLOTS)])
    pltpu.sync_copy(dc_hbm, dc_v)

    def issue(r, rb, sem):
        pltpu.async_copy(x2_hbm.at[idx_v.at[pl.ds(r * SLOTS, SLOTS)]], rb, sem)

    def wait(rb, sem):
        pltpu.make_async_copy(x2_hbm.at[pl.ds(0, SLOTS)], rb, sem).wait()

    for j in range(NBUF):
        issue(j, rbs[j], sems[j])

    def group(g, carry):
        r0 = g * NBUF
        for j in range(NBUF):
            wait(rbs[j], sems[j])
            _sc_row_compute(rbs[j], idx_v, ew_v, dc_v, w_v, bc_v, out_v, r0 + j)

            @pl.when(r0 + j + NBUF < RPW)
            def _next():
                issue(r0 + j + NBUF, rbs[j], sems[j])

        return carry

    lax.fori_loop(0, RPW // NBUF, group, 0)
    pltpu.sync_copy(out_v, out_hbm.at[pl.ds(base * D, RPW * D)])


@functools.cache
def _get_sc_agg():
    return functools.partial(
        pl.kernel,
        out_type=jax.ShapeDtypeStruct((N * D,), jnp.float32),
        mesh=plsc.VectorSubcoreMesh(core_axis_name="c", subcore_axis_name="s"),
        compiler_params=pltpu.CompilerParams(needs_layout_passes=False),
        scratch_types=[
            pltpu.VMEM((RPW * SLOTS + 16,), jnp.int32),
            pltpu.VMEM((RPW * SLOTS + 16,), jnp.float32),
            pltpu.VMEM((2 * N,), jnp.float32),
            pltpu.VMEM((SLOTS, D), jnp.float32),
            pltpu.VMEM((SLOTS, D), jnp.float32),
            pltpu.VMEM((SLOTS, D), jnp.float32),
            pltpu.VMEM((SLOTS, D), jnp.float32),
            pltpu.VMEM((SLOTS, D), jnp.float32),
            pltpu.VMEM((SLOTS, D), jnp.float32),
            pltpu.VMEM((SLOTS, D), jnp.float32),
            pltpu.VMEM((SLOTS, D), jnp.float32),
            pltpu.VMEM((16 + 48,), jnp.float32),
            pltpu.VMEM((16,), jnp.float32),
            pltpu.VMEM((RPW * D,), jnp.float32),
            pltpu.SemaphoreType.DMA,
            pltpu.SemaphoreType.DMA,
            pltpu.SemaphoreType.DMA,
            pltpu.SemaphoreType.DMA,
            pltpu.SemaphoreType.DMA,
            pltpu.SemaphoreType.DMA,
            pltpu.SemaphoreType.DMA,
            pltpu.SemaphoreType.DMA,
        ],
    )(_sc_agg_body)


def kernel(X, labels_cam):
    lab = labels_cam.astype(jnp.int32)
    ew, idx, dc = _topk_call(X, X, lab.reshape(N, 1), lab.reshape(1, N))
    x2 = jnp.concatenate([X, X], axis=0)
    out = _get_sc_agg()(x2, idx.reshape(-1), ew.reshape(-1), dc.reshape(-1))
    return out.reshape(N, D)


# speedup vs baseline: 2.1510x; 1.0008x over previous
import functools

import jax
import jax.numpy as jnp
from jax import lax
from jax.experimental import pallas as pl
from jax.experimental.pallas import tpu as pltpu
from jax.experimental.pallas import tpu_sc as plsc

N = 4096
D = 128
K1 = 30
K2 = 6
BETA1 = 0.2
BETA2 = 0.2
SCALE = 0.3
BLK = 256
NSTEPS = N // BLK
W1 = 32
W2 = 8
SLOTS = W1 + W2
NEG = -3.0


def _topk_body(xb_ref, xall_ref, labr_ref, labc_ref, ew_ref, idx_ref, dc_ref):
    step = pl.program_id(0)
    base = step * BLK
    xb = xb_ref[...]
    xall = xall_ref[...]
    sim = lax.dot_general(xb, xall, (((1,), (1,)), ((), ())),
                          preferred_element_type=jnp.float32)
    cols = lax.broadcasted_iota(jnp.int32, (BLK, N), 1)
    rid = base + lax.broadcasted_iota(jnp.int32, (BLK, 1), 0)
    diag = cols == rid

    def select(work0, k, width):
        wcols = lax.broadcasted_iota(jnp.int32, (BLK, width), 1)

        def body(t, carry):
            work, vals, idxs = carry
            m = jnp.max(work, axis=1, keepdims=True)
            eq = work == m
            sel = jnp.min(jnp.where(eq, cols, N), axis=1, keepdims=True)
            slot = wcols == t
            vals = jnp.where(slot, m, vals)
            idxs = jnp.where(slot, sel, idxs)
            work = jnp.where(eq, NEG, work)
            return work, vals, idxs

        vals0 = jnp.full((BLK, width), -1e30, jnp.float32)
        idxs0 = jnp.zeros((BLK, width), jnp.int32)
        return lax.fori_loop(0, k, body, (work0, vals0, idxs0))

    work1, vals1, idx1 = select(sim, K1, W1)
    es = jnp.exp(sim / BETA1)
    e1 = jnp.where(work1 == NEG, es, 0.0)
    rowsum1 = jnp.sum(e1, axis=1, keepdims=True)
    ew1 = ((1.0 - SCALE) * lax.rsqrt(rowsum1)) * jnp.exp(vals1 / BETA1)

    same_cam = labr_ref[...] == labc_ref[...]
    mask2 = same_cam & (~diag)
    sim2 = jnp.where(mask2, -2.0, sim)
    work2, vals2, idx2 = select(sim2, K2, W2)
    e2 = jnp.where(work2 == NEG,
                   jnp.where(mask2, jnp.float32(4.5399929762484854e-05), es),
                   0.0)
    e2 = jnp.where(diag, es, e2)
    rowsum2 = jnp.sum(e2, axis=1, keepdims=True)
    dr2 = SCALE * lax.rsqrt(rowsum2)
    ew2 = dr2 * jnp.exp(vals2 / BETA2)
    dvec = jnp.sum(jnp.where(diag, sim, 0.0), axis=1, keepdims=True)
    has_diag = jnp.any(idx2[:, :K2] == rid, axis=1, keepdims=True)
    ew_diag = jnp.where(has_diag, 0.0, dr2 * jnp.exp(dvec / BETA2))
    ew2_full = jnp.concatenate(
        [ew2[:, :K2], ew_diag, jnp.zeros((BLK, W2 - K2 - 1), jnp.float32)], axis=1)
    idx2_full = jnp.concatenate(
        [idx2[:, :K2], rid, jnp.zeros((BLK, W2 - K2 - 1), jnp.int32)], axis=1) + N

    ew_ref[...] = jnp.concatenate([ew1, ew2_full], axis=1)
    idx_ref[...] = jnp.concatenate([idx1, idx2_full], axis=1)

    cp = jnp.concatenate([jnp.sum(e1, axis=0, keepdims=True),
                          jnp.sum(e2, axis=0, keepdims=True)], axis=0)

    @pl.when(step == 0)
    def _init():
        dc_ref[...] = cp

    @pl.when(step != 0)
    def _acc():
        dc_ref[...] += cp

    @pl.when(step == NSTEPS - 1)
    def _fin():
        dc_ref[...] = lax.rsqrt(dc_ref[...])


_topk_call = pl.pallas_call(
    _topk_body,
    grid=(NSTEPS,),
    in_specs=[
        pl.BlockSpec((BLK, D), lambda i: (i, 0)),
        pl.BlockSpec((N, D), lambda i: (0, 0)),
        pl.BlockSpec((BLK, 1), lambda i: (i, 0)),
        pl.BlockSpec((1, N), lambda i: (0, 0)),
    ],
    out_specs=[
        pl.BlockSpec((BLK, SLOTS), lambda i: (i, 0)),
        pl.BlockSpec((BLK, SLOTS), lambda i: (i, 0)),
        pl.BlockSpec((2, N), lambda i: (0, 0)),
    ],
    out_shape=[
        jax.ShapeDtypeStruct((N, SLOTS), jnp.float32),
        jax.ShapeDtypeStruct((N, SLOTS), jnp.int32),
        jax.ShapeDtypeStruct((2, N), jnp.float32),
    ],
)


NW = 32
RPW = N // NW


def _sc_row_compute(rb, idx_v, ew_v, dc_v, w_v, bc_v, out_v, r):
    for c in range((SLOTS + 15) // 16):
        idxc = idx_v[pl.ds(r * SLOTS + c * 16, 16)]
        ewc = ew_v[pl.ds(r * SLOTS + c * 16, 16)]
        valid = SLOTS - c * 16
        if valid >= 16:
            dcg = plsc.load_gather(dc_v, [idxc])
        else:
            lanes = lax.broadcasted_iota(jnp.int32, (16,), 0)
            dcg = plsc.load_gather(dc_v, [idxc], mask=lanes < valid)
        w_v[pl.ds((c + 1) * 16, 16)] = ewc * dcg
    accs = [jnp.zeros((16,), jnp.float32) for _ in range(D // 16)]
    for k in range(SLOTS):
        wb = plsc.load_gather(w_v, [jnp.full((16,), 16 + k, jnp.int32)])
        for d in range(D // 16):
            accs[d] = accs[d] + wb * rb[k, pl.ds(d * 16, 16)]
    sq = accs[0] * accs[0]
    for d in range(1, D // 16):
        sq = sq + accs[d] * accs[d]
    bc_v[...] = jnp.cumsum(sq)
    tot = plsc.load_gather(bc_v, [jnp.full((16,), 15, jnp.int32)])
    yi = jnp.int32(0x5F3759DF) - lax.shift_right_logical(
        plsc.bitcast(tot, jnp.int32), 1)
    y = plsc.bitcast(yi, jnp.float32)
    for _ in range(3):
        y = y * (1.5 - 0.5 * tot * y * y)
    for d in range(D // 16):
        out_v[pl.ds(r * D + d * 16, 16)] = accs[d] * y


NBUF = 8


def _sc_agg_body(x2_hbm, idx_hbm, ew_hbm, dc_hbm, out_hbm,
                 idx_v, ew_v, dc_v, rb0, rb1, rb2, rb3, rb4, rb5, rb6, rb7,
                 w_v, bc_v, out_v,
                 sem0, sem1, sem2, sem3, sem4, sem5, sem6, sem7):
    rbs = (rb0, rb1, rb2, rb3, rb4, rb5, rb6, rb7)
    sems = (sem0, sem1, sem2, sem3, sem4, sem5, sem6, sem7)
    wid = lax.axis_index("s") * 2 + lax.axis_index("c")
    base = wid * RPW
    pltpu.sync_copy(idx_hbm.at[pl.ds(base * SLOTS, RPW * SLOTS)],
                    idx_v.at[pl.ds(0, RPW * SLOTS)])
    pltpu.sync_copy(ew_hbm.at[pl.ds(base * SLOTS, RPW * SLOTS)],
                    ew_v.at[pl.ds(0, RPW * SLOTS)])
    pltpu.sync_copy(dc_hbm, dc_v)

    def issue(r, rb, sem):
        pltpu.async_copy(x2_hbm.at[idx_v.at[pl.ds(r * SLOTS, SLOTS)]], rb, sem)

    def wait(rb, sem):
        pltpu.make_async_copy(x2_hbm.at[pl.ds(0, SLOTS)], rb, sem).wait()

    for j in range(NBUF):
        issue(j, rbs[j], sems[j])

    def group(g, carry):
        r0 = g * NBUF
        for j in range(NBUF):
            wait(rbs[j], sems[j])
            _sc_row_compute(rbs[j], idx_v, ew_v, dc_v, w_v, bc_v, out_v, r0 + j)

            @pl.when(r0 + j + NBUF < RPW)
            def _next():
                issue(r0 + j + NBUF, rbs[j], sems[j])

        return carry

    lax.fori_loop(0, RPW // NBUF, group, 0)
    pltpu.sync_copy(out_v, out_hbm.at[pl.ds(base * D, RPW * D)])


@functools.cache
def _get_sc_agg():
    return functools.partial(
        pl.kernel,
        out_type=jax.ShapeDtypeStruct((N * D,), jnp.float32),
        mesh=plsc.VectorSubcoreMesh(core_axis_name="c", subcore_axis_name="s"),
        compiler_params=pltpu.CompilerParams(needs_layout_passes=False),
        scratch_types=[
            pltpu.VMEM((RPW * SLOTS + 16,), jnp.int32),
            pltpu.VMEM((RPW * SLOTS + 16,), jnp.float32),
            pltpu.VMEM((2 * N,), jnp.float32),
            pltpu.VMEM((SLOTS, D), jnp.float32),
            pltpu.VMEM((SLOTS, D), jnp.float32),
            pltpu.VMEM((SLOTS, D), jnp.float32),
            pltpu.VMEM((SLOTS, D), jnp.float32),
            pltpu.VMEM((SLOTS, D), jnp.float32),
            pltpu.VMEM((SLOTS, D), jnp.float32),
            pltpu.VMEM((SLOTS, D), jnp.float32),
            pltpu.VMEM((SLOTS, D), jnp.float32),
            pltpu.VMEM((16 + 48,), jnp.float32),
            pltpu.VMEM((16,), jnp.float32),
            pltpu.VMEM((RPW * D,), jnp.float32),
            pltpu.SemaphoreType.DMA,
            pltpu.SemaphoreType.DMA,
            pltpu.SemaphoreType.DMA,
            pltpu.SemaphoreType.DMA,
            pltpu.SemaphoreType.DMA,
            pltpu.SemaphoreType.DMA,
            pltpu.SemaphoreType.DMA,
            pltpu.SemaphoreType.DMA,
        ],
    )(_sc_agg_body)


def kernel(X, labels_cam):
    lab = labels_cam.astype(jnp.int32)
    ew, idx, dc = _topk_call(X, X, lab.reshape(N, 1), lab.reshape(1, N))
    x2 = jnp.concatenate([X, X], axis=0)
    out = _get_sc_agg()(x2, idx.reshape(-1), ew.reshape(-1), dc.reshape(-1))
    return out.reshape(N, D)
